# baseline (speedup 1.0000x reference)
import functools

import jax
import jax.numpy as jnp
from jax import lax
from jax.experimental import pallas as pl
from jax.experimental.pallas import tpu as pltpu
from jax.experimental.pallas import tpu_sc as plsc

_INFO = plsc.get_sparse_core_info()
_NC = _INFO.num_cores
_NS = _INFO.num_subcores
_NL = _INFO.num_lanes
_NW = _NC * _NS

_BATCH = 16384
_DIM = 64
_BPW = _BATCH // _NW
_CHUNK = 128
_NCHUNK = _BPW // _CHUNK

_ENT = 1000000
_REL = 1000


def _rsqrt(x):
    xi = plsc.bitcast(x, jnp.int32)
    y = plsc.bitcast(jnp.int32(0x5F3759DF) - (xi >> 1), jnp.float32)
    for _ in range(3):
        y = y * (1.5 - 0.5 * x * y * y)
    return y


def _body(bh, bt, br, entf, relf, out, idx_h, idx_t, idx_r,
          ph, pt, pr, hb, tb, rb, out_v, sem_i, s0, s1):
    wid = lax.axis_index("s") * _NC + lax.axis_index("c")
    base = wid * _BPW

    ci = [pltpu.async_copy(src.at[pl.ds(base, _BPW)], dst, sem_i)
          for src, dst in ((bh, idx_h), (bt, idx_t), (br, idx_r))]
    for cp in ci:
        cp.wait()

    zf = jnp.zeros((_NL,), jnp.float32)
    sems = (s0, s1)

    def build(c, buf):
        def grp(g, _):
            s = pl.ds(c * _CHUNK + g * _NL, _NL)
            ihv = idx_h[s]
            itv = idx_t[s]
            irv = idx_r[s]

            def feat(j, _):
                d = pl.ds(j * _CHUNK + g * _NL, _NL)
                ph[buf, d] = ihv + j * _ENT
                pt[buf, d] = itv + j * _ENT
                pr[buf, d] = irv + j * _REL
                return 0

            lax.fori_loop(0, _DIM, feat, 0)
            return 0

        lax.fori_loop(0, _CHUNK // _NL, grp, 0)

    def fire(buf):
        return [
            pltpu.async_copy(entf.at[ph.at[buf]], hb.at[buf], sems[buf]),
            pltpu.async_copy(entf.at[pt.at[buf]], tb.at[buf], sems[buf]),
            pltpu.async_copy(relf.at[pr.at[buf]], rb.at[buf], sems[buf]),
        ]

    build(0, 0)
    pend = fire(0)
    for c in range(_NCHUNK):
        cur = c % 2
        if c + 1 < _NCHUNK:
            build(c + 1, 1 - cur)
        for cp in pend:
            cp.wait()
        if c + 1 < _NCHUNK:
            pend = fire(1 - cur)
        hc, tc, rc = hb.at[cur], tb.at[cur], rb.at[cur]

        def group(gi, _, hc=hc, tc=tc, rc=rc, c=c):

            def pass_a(jb, carry):
                hs, rs, ts = carry
                for jo in range(8):
                    j = jb * 8 + jo
                    d = pl.ds(j * _CHUNK + gi * _NL, _NL)
                    hj = hc[d]
                    tj = tc[d]
                    rj = rc[d]
                    hs = hs + hj * hj
                    ts = ts + tj * tj
                    rs = rs + rj * rj
                return hs, rs, ts

            hs, rs, ts = lax.fori_loop(0, _DIM // 8, pass_a, (zf, zf, zf))
            ih = _rsqrt(jnp.maximum(hs, 1e-24))
            ir = _rsqrt(jnp.maximum(rs, 1e-24))
            it = _rsqrt(jnp.maximum(ts, 1e-24))

            def pass_b(jb, score):
                for jo in range(8):
                    j = jb * 8 + jo
                    d = pl.ds(j * _CHUNK + gi * _NL, _NL)
                    hj = hc[d]
                    tj = tc[d]
                    rj = rc[d]
                    score = score + jnp.abs(hj * ih + rj * ir - tj * it)
                return score

            score = lax.fori_loop(0, _DIM // 8, pass_b, zf)
            out_v[pl.ds(c * _CHUNK + gi * _NL, _NL)] = score
            return 0

        lax.fori_loop(0, _CHUNK // _NL, group, 0)

    pltpu.sync_copy(out_v, out.at[pl.ds(base, _BPW)])


def kernel(batch_h, batch_t, batch_r, ent_emb, rel_emb):
    mesh = plsc.VectorSubcoreMesh(core_axis_name="c", subcore_axis_name="s")
    f = functools.partial(
        pl.kernel,
        mesh=mesh,
        compiler_params=pltpu.CompilerParams(
            needs_layout_passes=False, use_tc_tiling_on_sc=False),
        out_type=jax.ShapeDtypeStruct((_BATCH,), jnp.float32),
        scratch_types=[
            pltpu.VMEM((_BPW,), jnp.int32),
            pltpu.VMEM((_BPW,), jnp.int32),
            pltpu.VMEM((_BPW,), jnp.int32),
            pltpu.VMEM((2, _DIM * _CHUNK), jnp.int32),
            pltpu.VMEM((2, _DIM * _CHUNK), jnp.int32),
            pltpu.VMEM((2, _DIM * _CHUNK), jnp.int32),
            pltpu.VMEM((2, _DIM * _CHUNK), jnp.float32),
            pltpu.VMEM((2, _DIM * _CHUNK), jnp.float32),
            pltpu.VMEM((2, _DIM * _CHUNK), jnp.float32),
            pltpu.VMEM((_BPW,), jnp.float32),
            pltpu.SemaphoreType.DMA,
            pltpu.SemaphoreType.DMA,
            pltpu.SemaphoreType.DMA,
        ],
    )(_body)
    return f(batch_h, batch_t, batch_r,
             ent_emb.T.reshape(-1), rel_emb.T.reshape(-1))
